# baseline (speedup 1.0000x reference)
import jax
import jax.numpy as jnp
from jax import lax
from jax.experimental import pallas as pl
from jax.experimental.pallas import tpu as pltpu
from jax.experimental.pallas import tpu_sc as plsc

NS = 16
NC = 2
CHUNK = 80
KP = 2
NBUF = 2 * KP
EPS = 1e-5
BLK = 1000


def _row_split(N):
    rps = ((N + NS - 1) // NS + 7) // 8 * 8
    return rps, N - (NS - 1) * rps


def _agg_pass(h_hbm, src_h, dst_h, acc, src_vs, dst_vs, rows_vs, sems,
              e0, n_chunks):
    sem_i, sem_g, sem_s = sems
    pair = 2 * KP
    npairs = n_chunks // pair
    tail = n_chunks - npairs * pair

    def idx_copy(s, cbase):
        ds = []
        for b in range(KP):
            i = s * KP + b
            cb = cbase + b * CHUNK
            ds.append(pltpu.async_copy(
                src_h.at[pl.ds(cb, CHUNK)], src_vs[i], sem_i))
            ds.append(pltpu.async_copy(
                dst_h.at[pl.ds(cb, CHUNK)], dst_vs[i], sem_i))
        for d in ds:
            d.wait()

    def gath(s):
        for b in range(KP):
            i = s * KP + b
            pltpu.async_copy(h_hbm.at[src_vs[i]], rows_vs[i], sem_g)

    def scat(s):
        ds = []
        for b in range(KP):
            i = s * KP + b
            pltpu.make_async_copy(h_hbm.at[src_vs[i]], rows_vs[i], sem_g).wait()
            ds.append(pltpu.async_copy(
                rows_vs[i], acc.at[dst_vs[i]], sem_s, add=True))
        for d in ds:
            d.wait()

    if npairs > 0:
        idx_copy(0, e0)
        gath(0)

        def pair_body(p, carry):
            base_a = e0 + p * pair * CHUNK
            idx_copy(1, base_a + KP * CHUNK)
            gath(1)
            scat(0)

            @pl.when(p + 1 < npairs)
            def _():
                idx_copy(0, base_a + pair * CHUNK)
                gath(0)

            scat(1)
            return carry

        lax.fori_loop(0, npairs, pair_body, 0)

    for t in range(tail):
        cb = e0 + (npairs * pair + t) * CHUNK
        pltpu.sync_copy(src_h.at[pl.ds(cb, CHUNK)], src_vs[t])
        pltpu.sync_copy(dst_h.at[pl.ds(cb, CHUNK)], dst_vs[t])
        pltpu.async_copy(h_hbm.at[src_vs[t]], rows_vs[t], sem_g).wait()
        pltpu.async_copy(rows_vs[t], acc.at[dst_vs[t]], sem_s, add=True).wait()


def _deg_pass(dst_h, ones_v, acc, dst_vs, sems, e0, n_chunks):
    sem_i, _, sem_s = sems
    kb = len(dst_vs)
    ngr = n_chunks // kb
    tail = n_chunks - ngr * kb

    def grp(gi, carry):
        base = e0 + gi * kb * CHUNK
        idx_d = [pltpu.async_copy(
            dst_h.at[pl.ds(base + b * CHUNK, CHUNK)], dst_vs[b], sem_i)
            for b in range(kb)]
        s_d = []
        for b in range(kb):
            idx_d[b].wait()
            s_d.append(pltpu.async_copy(
                ones_v, acc.at[dst_vs[b]], sem_s, add=True))
        for d in s_d:
            d.wait()
        return carry

    lax.fori_loop(0, ngr, grp, 0)
    for t in range(tail):
        cb = e0 + (ngr * kb + t) * CHUNK
        pltpu.sync_copy(dst_h.at[pl.ds(cb, CHUNK)], dst_vs[t])
        pltpu.async_copy(ones_v, acc.at[dst_vs[t]], sem_s, add=True).wait()


def _make_seg0(N, E, D):
    E2 = E // NC
    per_sub = E2 // NS
    n_chunks = per_sub // CHUNK
    rps, rlast = _row_split(N)
    mesh = plsc.VectorSubcoreMesh(core_axis_name="c", subcore_axis_name="s")

    out_type = [jax.ShapeDtypeStruct((N, D), jnp.float32) for _ in range(4)]
    scratch = (
        [pltpu.VMEM_SHARED((N, D), jnp.float32)]
        + [pltpu.VMEM((CHUNK,), jnp.int32) for _ in range(NBUF)]
        + [pltpu.VMEM((CHUNK,), jnp.int32) for _ in range(NBUF)]
        + [pltpu.VMEM((CHUNK, D), jnp.float32) for _ in range(NBUF)]
        + [pltpu.SemaphoreType.DMA] * 3
    )

    def body(x_h, src_h, dst_h, zeros_h, ones_h, p0, p1, d0, d1, acc, *rest):
        src_vs = rest[0:NBUF]
        dst_vs = rest[NBUF:2 * NBUF]
        rows_vs = rest[2 * NBUF:3 * NBUF]
        sems = rest[3 * NBUF:3 * NBUF + 3]
        cid = lax.axis_index("c")
        sid = lax.axis_index("s")

        def with_rows(fn):
            @pl.when(sid < NS - 1)
            def _():
                fn(sid * rps, rps)

            @pl.when(sid == NS - 1)
            def _():
                fn((NS - 1) * rps, rlast)

        def run(p_out, d_out):
            e0 = cid * E2 + sid * per_sub

            with_rows(lambda r0, nr: pltpu.sync_copy(
                zeros_h.at[pl.ds(0, nr)], acc.at[pl.ds(r0, nr)]))
            plsc.subcore_barrier()
            _agg_pass(x_h, src_h, dst_h, acc, src_vs, dst_vs, rows_vs, sems,
                      e0, n_chunks)
            plsc.subcore_barrier()

            def outA(r0, nr):
                pltpu.sync_copy(acc.at[pl.ds(r0, nr)], p_out.at[pl.ds(r0, nr)])
                pltpu.sync_copy(zeros_h.at[pl.ds(0, nr)], acc.at[pl.ds(r0, nr)])

            with_rows(outA)
            pltpu.sync_copy(ones_h, rows_vs[0])
            plsc.subcore_barrier()

            _deg_pass(dst_h, rows_vs[0], acc, dst_vs, sems, e0, n_chunks)
            plsc.subcore_barrier()
            with_rows(lambda r0, nr: pltpu.sync_copy(
                acc.at[pl.ds(r0, nr)], d_out.at[pl.ds(r0, nr)]))

        @pl.when(cid == 0)
        def _():
            run(p0, d0)

        @pl.when(cid == 1)
        def _():
            run(p1, d1)

    return pl.kernel(body, out_type=out_type, mesh=mesh, scratch_types=scratch)


def _make_segsum(N, E, Dh):
    per_sub = E // NS
    n_chunks = per_sub // CHUNK
    rps, rlast = _row_split(N)
    mesh = plsc.VectorSubcoreMesh(core_axis_name="c", subcore_axis_name="s")

    out_type = [jax.ShapeDtypeStruct((N, Dh), jnp.float32),
                jax.ShapeDtypeStruct((N, Dh), jnp.float32)]
    scratch = (
        [pltpu.VMEM_SHARED((N, Dh), jnp.float32)]
        + [pltpu.VMEM((CHUNK,), jnp.int32) for _ in range(NBUF)]
        + [pltpu.VMEM((CHUNK,), jnp.int32) for _ in range(NBUF)]
        + [pltpu.VMEM((CHUNK, Dh), jnp.float32) for _ in range(NBUF)]
        + [pltpu.SemaphoreType.DMA] * 3
    )

    def body(h0, h1, src_h, dst_h, zeros_h, agg0, agg1, acc, *rest):
        src_vs = rest[0:NBUF]
        dst_vs = rest[NBUF:2 * NBUF]
        rows_vs = rest[2 * NBUF:3 * NBUF]
        sems = rest[3 * NBUF:3 * NBUF + 3]
        cid = lax.axis_index("c")
        sid = lax.axis_index("s")

        def with_rows(fn):
            @pl.when(sid < NS - 1)
            def _():
                fn(sid * rps, rps)

            @pl.when(sid == NS - 1)
            def _():
                fn((NS - 1) * rps, rlast)

        def run(h_hbm, agg_hbm):
            with_rows(lambda r0, nr: pltpu.sync_copy(
                zeros_h.at[pl.ds(0, nr)], acc.at[pl.ds(r0, nr)]))
            plsc.subcore_barrier()
            _agg_pass(h_hbm, src_h, dst_h, acc, src_vs, dst_vs, rows_vs, sems,
                      sid * per_sub, n_chunks)
            plsc.subcore_barrier()
            with_rows(lambda r0, nr: pltpu.sync_copy(
                acc.at[pl.ds(r0, nr)], agg_hbm.at[pl.ds(r0, nr)]))

        @pl.when(cid == 0)
        def _():
            run(h0, agg0)

        @pl.when(cid == 1)
        def _():
            run(h1, agg1)

    return pl.kernel(body, out_type=out_type, mesh=mesh, scratch_types=scratch)


def _make_layer(N, H, C, lidx, first, last):
    Dh = H // 2
    G = N // BLK

    def body(*refs):
        if first:
            (p0, p1, d0, d1, x, wl, wr, b, g, be, wjk, prev) = refs[:12]
            outs = refs[12:-5]
            z_scr, inv_scr, s1, s2, sc2 = refs[-5:]
        else:
            (a0, a1, inv_in, h0, h1, wl, wr, b, g, be, wjk, prev) = refs[:12]
            outs = refs[12:-4]
            z_scr, s1, s2, sc2 = refs[-4:]
        i = pl.program_id(0)

        @pl.when(i < G)
        def _pass1():
            if first:
                deg = d0[...][:, 0:1] + d1[...][:, 0:1]
                inv = 1.0 / jnp.maximum(deg, 1.0)
                inv_scr[pl.ds(i * BLK, BLK), :] = inv
                mean = (p0[...] + p1[...]) * inv
                hself = x[...]
            else:
                mean = jnp.concatenate([a0[...], a1[...]], axis=1) * inv_in[...]
                hself = jnp.concatenate([h0[...], h1[...]], axis=1)
            z = (jnp.dot(mean.astype(jnp.bfloat16), wl[...],
                         preferred_element_type=jnp.float32)
                 + jnp.dot(hself.astype(jnp.bfloat16), wr[...],
                           preferred_element_type=jnp.float32)
                 + b[...])
            z_scr[pl.ds(i * BLK, BLK), :] = z
            ps1 = jnp.sum(z, axis=0, keepdims=True)
            ps2 = jnp.sum(z * z, axis=0, keepdims=True)

            @pl.when(i == 0)
            def _():
                s1[...] = ps1
                s2[...] = ps2

            @pl.when(i > 0)
            def _():
                s1[...] = s1[...] + ps1
                s2[...] = s2[...] + ps2

        @pl.when(i >= G)
        def _pass2():
            @pl.when(i == G)
            def _():
                mu = s1[...] / N
                var = s2[...] / N - mu * mu
                scale = lax.rsqrt(var + EPS) * g[...]
                sc2[0:1, :] = scale
                sc2[1:2, :] = be[...] - mu * scale

            z = z_scr[pl.ds((i - G) * BLK, BLK), :]
            hn = jnp.maximum(z * sc2[0:1, :] + sc2[1:2, :], 0.0)
            contrib = jnp.dot(hn.astype(jnp.bfloat16), wjk[...],
                              preferred_element_type=jnp.float32)
            if first:
                out_r = outs[2]
                out_r[...] = prev[...] + contrib
                outs[3][...] = inv_scr[pl.ds((i - G) * BLK, BLK), :]
            else:
                out_r = outs[2] if not last else outs[0]
                out_r[...] = prev[...] + contrib
            if not last:
                outs[0][...] = hn[:, :Dh]
                outs[1][...] = hn[:, Dh:]

    def p1map(i):
        return (jnp.where(i < G, i, 0), 0)

    def p2map(i):
        return (jnp.where(i >= G, i - G, 0), 0)

    row = pl.BlockSpec((BLK, Dh if not first else 128), p1map)
    Din = 128 if first else H
    const2 = lambda s: pl.BlockSpec(s, lambda i: (0, 0))
    wjk_spec = pl.BlockSpec((H, C), lambda i: (lidx, 0))
    prev_spec = (pl.BlockSpec((BLK, C), p2map) if not first
                 else pl.BlockSpec((1, C), lambda i: (0, 0)))
    if first:
        in_specs = [pl.BlockSpec((BLK, 128), p1map)] * 5
    else:
        in_specs = [row, row, pl.BlockSpec((BLK, 1), p1map), row, row]
    in_specs += [const2((Din, H)), const2((Din, H)), const2((1, H)),
                 const2((1, H)), const2((1, H)), wjk_spec, prev_spec]

    out_specs = []
    out_shape = []
    if not last:
        out_specs += [pl.BlockSpec((BLK, Dh), p2map),
                      pl.BlockSpec((BLK, Dh), p2map)]
        out_shape += [jax.ShapeDtypeStruct((N, Dh), jnp.float32),
                      jax.ShapeDtypeStruct((N, Dh), jnp.float32)]
    out_specs += [pl.BlockSpec((BLK, C), p2map)]
    out_shape += [jax.ShapeDtypeStruct((N, C), jnp.float32)]
    if first:
        out_specs += [pl.BlockSpec((BLK, 1), p2map)]
        out_shape += [jax.ShapeDtypeStruct((N, 1), jnp.float32)]

    scratch = [pltpu.VMEM((N, H), jnp.float32)]
    if first:
        scratch += [pltpu.VMEM((N, 1), jnp.float32)]
    scratch += [pltpu.VMEM((1, H), jnp.float32),
                pltpu.VMEM((1, H), jnp.float32),
                pltpu.VMEM((2, H), jnp.float32)]

    return pl.pallas_call(
        body,
        grid=(2 * G,),
        in_specs=in_specs,
        out_specs=out_specs,
        out_shape=out_shape,
        scratch_shapes=scratch,
    )


def kernel(x, edge_index, Wl0, Wr0, b0, g0, be0, Wl1, Wr1, b1, g1, be1,
           Wl2, Wr2, b2, g2, be2, Wlin, blin):
    N, IN = x.shape
    E = edge_index.shape[1]
    H = Wl0.shape[1]
    C = Wlin.shape[1]
    Dh = H // 2

    src = edge_index[0]
    dst = edge_index[1]
    rps, _ = _row_split(N)
    zeros = jnp.zeros((rps, IN), jnp.float32)
    ones = jnp.ones((CHUNK, IN), jnp.float32)
    bf = jnp.bfloat16
    Wl0, Wr0, Wl1, Wr1, Wl2, Wr2, Wlin = (
        w.astype(bf) for w in (Wl0, Wr0, Wl1, Wr1, Wl2, Wr2, Wlin))

    seg0 = _make_seg0(N, E, IN)
    segH = _make_segsum(N, E, Dh)
    lay0 = _make_layer(N, H, C, 0, True, False)
    lay1 = _make_layer(N, H, C, 1, False, False)
    lay2 = _make_layer(N, H, C, 2, False, True)

    p0, p1, d0, d1 = seg0(x, src, dst, zeros, ones)
    h0, h1, out, inv = lay0(p0, p1, d0, d1, x, Wl0, Wr0, b0.reshape(1, H),
                            g0.reshape(1, H), be0.reshape(1, H), Wlin,
                            blin.reshape(1, C))
    a0, a1 = segH(h0, h1, src, dst, zeros)
    h0, h1, out = lay1(a0, a1, inv, h0, h1, Wl1, Wr1, b1.reshape(1, H),
                       g1.reshape(1, H), be1.reshape(1, H), Wlin, out)
    a0, a1 = segH(h0, h1, src, dst, zeros)
    (out,) = lay2(a0, a1, inv, h0, h1, Wl2, Wr2, b2.reshape(1, H),
                  g2.reshape(1, H), be2.reshape(1, H), Wlin, out)
    return out


# speedup vs baseline: 1.0212x; 1.0212x over previous
import jax
import jax.numpy as jnp
from jax import lax
from jax.experimental import pallas as pl
from jax.experimental.pallas import tpu as pltpu
from jax.experimental.pallas import tpu_sc as plsc

NS = 16
NC = 2
CHUNK = 80
KP = 2
NBUF = 2 * KP
EPS = 1e-5
BLK = 2000


def _row_split(N):
    rps = ((N + NS - 1) // NS + 7) // 8 * 8
    return rps, N - (NS - 1) * rps


def _agg_pass(h_hbm, src_h, dst_h, acc, src_vs, dst_vs, rows_vs, sems,
              e0, n_chunks):
    sem_i, sem_g, sem_s = sems
    pair = 2 * KP
    npairs = n_chunks // pair
    tail = n_chunks - npairs * pair

    def idx_copy(s, cbase):
        ds = []
        for b in range(KP):
            i = s * KP + b
            cb = cbase + b * CHUNK
            ds.append(pltpu.async_copy(
                src_h.at[pl.ds(cb, CHUNK)], src_vs[i], sem_i))
            ds.append(pltpu.async_copy(
                dst_h.at[pl.ds(cb, CHUNK)], dst_vs[i], sem_i))
        for d in ds:
            d.wait()

    def gath(s):
        for b in range(KP):
            i = s * KP + b
            pltpu.async_copy(h_hbm.at[src_vs[i]], rows_vs[i], sem_g)

    def scat(s):
        ds = []
        for b in range(KP):
            i = s * KP + b
            pltpu.make_async_copy(h_hbm.at[src_vs[i]], rows_vs[i], sem_g).wait()
            ds.append(pltpu.async_copy(
                rows_vs[i], acc.at[dst_vs[i]], sem_s, add=True))
        for d in ds:
            d.wait()

    if npairs > 0:
        idx_copy(0, e0)
        gath(0)

        def pair_body(p, carry):
            base_a = e0 + p * pair * CHUNK
            idx_copy(1, base_a + KP * CHUNK)
            gath(1)
            scat(0)

            @pl.when(p + 1 < npairs)
            def _():
                idx_copy(0, base_a + pair * CHUNK)
                gath(0)

            scat(1)
            return carry

        lax.fori_loop(0, npairs, pair_body, 0)

    for t in range(tail):
        cb = e0 + (npairs * pair + t) * CHUNK
        pltpu.sync_copy(src_h.at[pl.ds(cb, CHUNK)], src_vs[t])
        pltpu.sync_copy(dst_h.at[pl.ds(cb, CHUNK)], dst_vs[t])
        pltpu.async_copy(h_hbm.at[src_vs[t]], rows_vs[t], sem_g).wait()
        pltpu.async_copy(rows_vs[t], acc.at[dst_vs[t]], sem_s, add=True).wait()


def _deg_pass(dst_h, ones_v, acc, dst_vs, sems, e0, n_chunks):
    sem_i, _, sem_s = sems
    kb = len(dst_vs)
    ngr = n_chunks // kb
    tail = n_chunks - ngr * kb

    def grp(gi, carry):
        base = e0 + gi * kb * CHUNK
        idx_d = [pltpu.async_copy(
            dst_h.at[pl.ds(base + b * CHUNK, CHUNK)], dst_vs[b], sem_i)
            for b in range(kb)]
        s_d = []
        for b in range(kb):
            idx_d[b].wait()
            s_d.append(pltpu.async_copy(
                ones_v, acc.at[dst_vs[b]], sem_s, add=True))
        for d in s_d:
            d.wait()
        return carry

    lax.fori_loop(0, ngr, grp, 0)
    for t in range(tail):
        cb = e0 + (ngr * kb + t) * CHUNK
        pltpu.sync_copy(dst_h.at[pl.ds(cb, CHUNK)], dst_vs[t])
        pltpu.async_copy(ones_v, acc.at[dst_vs[t]], sem_s, add=True).wait()


def _make_seg0(N, E, D):
    E2 = E // NC
    per_sub = E2 // NS
    n_chunks = per_sub // CHUNK
    rps, rlast = _row_split(N)
    mesh = plsc.VectorSubcoreMesh(core_axis_name="c", subcore_axis_name="s")

    out_type = [jax.ShapeDtypeStruct((N, D), jnp.float32) for _ in range(4)]
    scratch = (
        [pltpu.VMEM_SHARED((N, D), jnp.float32)]
        + [pltpu.VMEM((CHUNK,), jnp.int32) for _ in range(NBUF)]
        + [pltpu.VMEM((CHUNK,), jnp.int32) for _ in range(NBUF)]
        + [pltpu.VMEM((CHUNK, D), jnp.float32) for _ in range(NBUF)]
        + [pltpu.SemaphoreType.DMA] * 3
    )

    def body(x_h, src_h, dst_h, zeros_h, ones_h, p0, p1, d0, d1, acc, *rest):
        src_vs = rest[0:NBUF]
        dst_vs = rest[NBUF:2 * NBUF]
        rows_vs = rest[2 * NBUF:3 * NBUF]
        sems = rest[3 * NBUF:3 * NBUF + 3]
        cid = lax.axis_index("c")
        sid = lax.axis_index("s")

        def with_rows(fn):
            @pl.when(sid < NS - 1)
            def _():
                fn(sid * rps, rps)

            @pl.when(sid == NS - 1)
            def _():
                fn((NS - 1) * rps, rlast)

        def run(p_out, d_out):
            e0 = cid * E2 + sid * per_sub

            with_rows(lambda r0, nr: pltpu.sync_copy(
                zeros_h.at[pl.ds(0, nr)], acc.at[pl.ds(r0, nr)]))
            plsc.subcore_barrier()
            _agg_pass(x_h, src_h, dst_h, acc, src_vs, dst_vs, rows_vs, sems,
                      e0, n_chunks)
            plsc.subcore_barrier()

            def outA(r0, nr):
                pltpu.sync_copy(acc.at[pl.ds(r0, nr)], p_out.at[pl.ds(r0, nr)])
                pltpu.sync_copy(zeros_h.at[pl.ds(0, nr)], acc.at[pl.ds(r0, nr)])

            with_rows(outA)
            pltpu.sync_copy(ones_h, rows_vs[0])
            plsc.subcore_barrier()

            _deg_pass(dst_h, rows_vs[0], acc, dst_vs, sems, e0, n_chunks)
            plsc.subcore_barrier()
            with_rows(lambda r0, nr: pltpu.sync_copy(
                acc.at[pl.ds(r0, nr)], d_out.at[pl.ds(r0, nr)]))

        @pl.when(cid == 0)
        def _():
            run(p0, d0)

        @pl.when(cid == 1)
        def _():
            run(p1, d1)

    return pl.kernel(body, out_type=out_type, mesh=mesh, scratch_types=scratch)


def _make_segsum(N, E, Dh):
    per_sub = E // NS
    n_chunks = per_sub // CHUNK
    rps, rlast = _row_split(N)
    mesh = plsc.VectorSubcoreMesh(core_axis_name="c", subcore_axis_name="s")

    out_type = [jax.ShapeDtypeStruct((N, Dh), jnp.float32),
                jax.ShapeDtypeStruct((N, Dh), jnp.float32)]
    scratch = (
        [pltpu.VMEM_SHARED((N, Dh), jnp.float32)]
        + [pltpu.VMEM((CHUNK,), jnp.int32) for _ in range(NBUF)]
        + [pltpu.VMEM((CHUNK,), jnp.int32) for _ in range(NBUF)]
        + [pltpu.VMEM((CHUNK, Dh), jnp.float32) for _ in range(NBUF)]
        + [pltpu.SemaphoreType.DMA] * 3
    )

    def body(h0, h1, src_h, dst_h, zeros_h, agg0, agg1, acc, *rest):
        src_vs = rest[0:NBUF]
        dst_vs = rest[NBUF:2 * NBUF]
        rows_vs = rest[2 * NBUF:3 * NBUF]
        sems = rest[3 * NBUF:3 * NBUF + 3]
        cid = lax.axis_index("c")
        sid = lax.axis_index("s")

        def with_rows(fn):
            @pl.when(sid < NS - 1)
            def _():
                fn(sid * rps, rps)

            @pl.when(sid == NS - 1)
            def _():
                fn((NS - 1) * rps, rlast)

        def run(h_hbm, agg_hbm):
            with_rows(lambda r0, nr: pltpu.sync_copy(
                zeros_h.at[pl.ds(0, nr)], acc.at[pl.ds(r0, nr)]))
            plsc.subcore_barrier()
            _agg_pass(h_hbm, src_h, dst_h, acc, src_vs, dst_vs, rows_vs, sems,
                      sid * per_sub, n_chunks)
            plsc.subcore_barrier()
            with_rows(lambda r0, nr: pltpu.sync_copy(
                acc.at[pl.ds(r0, nr)], agg_hbm.at[pl.ds(r0, nr)]))

        @pl.when(cid == 0)
        def _():
            run(h0, agg0)

        @pl.when(cid == 1)
        def _():
            run(h1, agg1)

    return pl.kernel(body, out_type=out_type, mesh=mesh, scratch_types=scratch)


def _make_layer(N, H, C, lidx, first, last):
    Dh = H // 2
    G = N // BLK

    def body(*refs):
        if first:
            (p0, p1, d0, d1, x, wl, wr, b, g, be, wjk, prev) = refs[:12]
            outs = refs[12:-5]
            z_scr, inv_scr, s1, s2, sc2 = refs[-5:]
        else:
            (a0, a1, inv_in, h0, h1, wl, wr, b, g, be, wjk, prev) = refs[:12]
            outs = refs[12:-4]
            z_scr, s1, s2, sc2 = refs[-4:]
        i = pl.program_id(0)

        @pl.when(i < G)
        def _pass1():
            if first:
                deg = d0[...][:, 0:1] + d1[...][:, 0:1]
                inv = 1.0 / jnp.maximum(deg, 1.0)
                inv_scr[pl.ds(i * BLK, BLK), :] = inv
                mean = (p0[...] + p1[...]) * inv
                hself = x[...]
            else:
                mean = jnp.concatenate([a0[...], a1[...]], axis=1) * inv_in[...]
                hself = jnp.concatenate([h0[...], h1[...]], axis=1)
            z = (jnp.dot(mean, wl[...], preferred_element_type=jnp.float32)
                 + jnp.dot(hself, wr[...], preferred_element_type=jnp.float32)
                 + b[...])
            z_scr[pl.ds(i * BLK, BLK), :] = z
            ps1 = jnp.sum(z, axis=0, keepdims=True)
            ps2 = jnp.sum(z * z, axis=0, keepdims=True)

            @pl.when(i == 0)
            def _():
                s1[...] = ps1
                s2[...] = ps2

            @pl.when(i > 0)
            def _():
                s1[...] = s1[...] + ps1
                s2[...] = s2[...] + ps2

        @pl.when(i >= G)
        def _pass2():
            @pl.when(i == G)
            def _():
                mu = s1[...] / N
                var = s2[...] / N - mu * mu
                scale = lax.rsqrt(var + EPS) * g[...]
                sc2[0:1, :] = scale
                sc2[1:2, :] = be[...] - mu * scale

            z = z_scr[pl.ds((i - G) * BLK, BLK), :]
            hn = jnp.maximum(z * sc2[0:1, :] + sc2[1:2, :], 0.0)
            contrib = jnp.dot(hn, wjk[...], preferred_element_type=jnp.float32)
            if first:
                out_r = outs[2]
                out_r[...] = prev[...] + contrib
                outs[3][...] = inv_scr[pl.ds((i - G) * BLK, BLK), :]
            else:
                out_r = outs[2] if not last else outs[0]
                out_r[...] = prev[...] + contrib
            if not last:
                outs[0][...] = hn[:, :Dh]
                outs[1][...] = hn[:, Dh:]

    def p1map(i):
        return (jnp.where(i < G, i, 0), 0)

    def p2map(i):
        return (jnp.where(i >= G, i - G, 0), 0)

    row = pl.BlockSpec((BLK, Dh if not first else 128), p1map)
    Din = 128 if first else H
    const2 = lambda s: pl.BlockSpec(s, lambda i: (0, 0))
    wjk_spec = pl.BlockSpec((H, C), lambda i: (lidx, 0))
    prev_spec = (pl.BlockSpec((BLK, C), p2map) if not first
                 else pl.BlockSpec((1, C), lambda i: (0, 0)))
    if first:
        in_specs = [pl.BlockSpec((BLK, 128), p1map)] * 5
    else:
        in_specs = [row, row, pl.BlockSpec((BLK, 1), p1map), row, row]
    in_specs += [const2((Din, H)), const2((Din, H)), const2((1, H)),
                 const2((1, H)), const2((1, H)), wjk_spec, prev_spec]

    out_specs = []
    out_shape = []
    if not last:
        out_specs += [pl.BlockSpec((BLK, Dh), p2map),
                      pl.BlockSpec((BLK, Dh), p2map)]
        out_shape += [jax.ShapeDtypeStruct((N, Dh), jnp.float32),
                      jax.ShapeDtypeStruct((N, Dh), jnp.float32)]
    out_specs += [pl.BlockSpec((BLK, C), p2map)]
    out_shape += [jax.ShapeDtypeStruct((N, C), jnp.float32)]
    if first:
        out_specs += [pl.BlockSpec((BLK, 1), p2map)]
        out_shape += [jax.ShapeDtypeStruct((N, 1), jnp.float32)]

    scratch = [pltpu.VMEM((N, H), jnp.float32)]
    if first:
        scratch += [pltpu.VMEM((N, 1), jnp.float32)]
    scratch += [pltpu.VMEM((1, H), jnp.float32),
                pltpu.VMEM((1, H), jnp.float32),
                pltpu.VMEM((2, H), jnp.float32)]

    return pl.pallas_call(
        body,
        grid=(2 * G,),
        in_specs=in_specs,
        out_specs=out_specs,
        out_shape=out_shape,
        scratch_shapes=scratch,
    )


def kernel(x, edge_index, Wl0, Wr0, b0, g0, be0, Wl1, Wr1, b1, g1, be1,
           Wl2, Wr2, b2, g2, be2, Wlin, blin):
    N, IN = x.shape
    E = edge_index.shape[1]
    H = Wl0.shape[1]
    C = Wlin.shape[1]
    Dh = H // 2

    src = edge_index[0]
    dst = edge_index[1]
    rps, _ = _row_split(N)
    zeros = jnp.zeros((rps, IN), jnp.float32)
    ones = jnp.ones((CHUNK, IN), jnp.float32)

    seg0 = _make_seg0(N, E, IN)
    segH = _make_segsum(N, E, Dh)
    lay0 = _make_layer(N, H, C, 0, True, False)
    lay1 = _make_layer(N, H, C, 1, False, False)
    lay2 = _make_layer(N, H, C, 2, False, True)

    p0, p1, d0, d1 = seg0(x, src, dst, zeros, ones)
    h0, h1, out, inv = lay0(p0, p1, d0, d1, x, Wl0, Wr0, b0.reshape(1, H),
                            g0.reshape(1, H), be0.reshape(1, H), Wlin,
                            blin.reshape(1, C))
    a0, a1 = segH(h0, h1, src, dst, zeros)
    h0, h1, out = lay1(a0, a1, inv, h0, h1, Wl1, Wr1, b1.reshape(1, H),
                       g1.reshape(1, H), be1.reshape(1, H), Wlin, out)
    a0, a1 = segH(h0, h1, src, dst, zeros)
    (out,) = lay2(a0, a1, inv, h0, h1, Wl2, Wr2, b2.reshape(1, H),
                  g2.reshape(1, H), be2.reshape(1, H), Wlin, out)
    return out
